# baseline (speedup 1.0000x reference)
import functools

import jax
import jax.numpy as jnp
from jax import lax
from jax.experimental import pallas as pl
from jax.experimental.pallas import tpu as pltpu
from jax.experimental.pallas import tpu_sc as plsc

N = 10000
M = 4
D = 128
E = 320000
ET = M * E

NT = 32
CHUNK = ET // NT
BROWS = 64
NB = 157
NBP = 160
RWORDS = BROWS * N
SLICE = RWORDS // 16
BINPAD = 4096
WA = 8000
WAP = 8064
WB = 496
WBUF = 512

_mesh = plsc.VectorSubcoreMesh(core_axis_name="c", subcore_axis_name="s")
_CP = pltpu.CompilerParams(needs_layout_passes=False)

BNM = 2000
BN2 = 1000


def _iota16():
    return lax.broadcasted_iota(jnp.int32, (16,), 0)


def _proj_kernel(z_ref, w1_ref, b1_ref, w2_ref, wsum_ref, beta_ref, *, ng):
    i = pl.program_id(0)
    h = jnp.dot(z_ref[...], w1_ref[...], preferred_element_type=jnp.float32)
    h = h + b1_ref[...]
    w = jnp.dot(h, w2_ref[...], preferred_element_type=jnp.float32)
    w = jnp.where(w >= 0, w, 0.01 * w)
    rid = lax.broadcasted_iota(jnp.int32, (BNM, M), 0)
    cid = lax.broadcasted_iota(jnp.int32, (BNM, M), 1)
    onehot = (rid % M == cid).astype(jnp.float32)
    part = jnp.sum(w * onehot, axis=0, keepdims=True)

    @pl.when(i == 0)
    def _():
        wsum_ref[...] = part

    @pl.when(i > 0)
    def _():
        wsum_ref[...] = wsum_ref[...] + part

    @pl.when(i == ng - 1)
    def _():
        acc = wsum_ref[...] / float(N)
        mx = jnp.max(acc)
        e = jnp.exp(acc - mx)
        beta_ref[...] = e / jnp.sum(e)


def _compute_beta(z, W1, b1, W2):
    zf = z.reshape(N * M, D)
    ng = (N * M) // BNM
    _, beta = pl.pallas_call(
        functools.partial(_proj_kernel, ng=ng),
        grid=(ng,),
        in_specs=[
            pl.BlockSpec((BNM, D), lambda i: (i, 0)),
            pl.BlockSpec((D, D), lambda i: (0, 0)),
            pl.BlockSpec((1, D), lambda i: (0, 0)),
            pl.BlockSpec((D, 1), lambda i: (0, 0)),
        ],
        out_specs=[
            pl.BlockSpec((1, M), lambda i: (0, 0)),
            pl.BlockSpec((1, M), lambda i: (0, 0)),
        ],
        out_shape=[
            jax.ShapeDtypeStruct((1, M), jnp.float32),
            jax.ShapeDtypeStruct((1, M), jnp.float32),
        ],
    )(zf, W1, b1.reshape(1, D), W2)
    return beta


def _zout_kernel(z_ref, beta_ref, out_ref):
    acc = beta_ref[0, 0] * z_ref[:, 0, :]
    for m in range(1, M):
        acc = acc + beta_ref[0, m] * z_ref[:, m, :]
    out_ref[...] = acc


def _compute_zout(z, beta):
    ng = N // BN2
    return pl.pallas_call(
        _zout_kernel,
        grid=(ng,),
        in_specs=[
            pl.BlockSpec((BN2, M, D), lambda i: (i, 0, 0)),
            pl.BlockSpec((1, M), lambda i: (0, 0)),
        ],
        out_specs=pl.BlockSpec((BN2, D), lambda i: (i, 0)),
        out_shape=jax.ShapeDtypeStruct((N, D), jnp.float32),
    )(z, beta)


@functools.partial(
    pl.kernel,
    mesh=_mesh,
    compiler_params=_CP,
    out_type=jax.ShapeDtypeStruct((NT, NBP), jnp.int32),
    scratch_types=[
        pltpu.VMEM((WA,), jnp.int32),
        pltpu.VMEM((NBP,), jnp.int32),
    ],
)
def _count_kernel(esrc_hbm, hist_hbm, src_v, hist_v):
    wid = lax.axis_index("s") * 2 + lax.axis_index("c")
    off = wid * CHUNK
    zero16 = jnp.zeros((16,), jnp.int32)
    for j in range(NBP // 16):
        hist_v[pl.ds(j * 16, 16)] = zero16

    def win(w, _):
        wo = pl.multiple_of(off + w * WA, 8)
        pltpu.sync_copy(esrc_hbm.at[pl.ds(wo, WA)], src_v)

        def vreg(j, _):
            jo = pl.multiple_of(j * 16, 16)
            s = src_v[pl.ds(jo, 16)]
            b = lax.shift_right_logical(s, 6)
            cnt, last = plsc.scan_count(b)
            plsc.addupdate_scatter(hist_v, [b], cnt.astype(jnp.int32),
                                   mask=last)
            return 0

        return lax.fori_loop(0, WA // 16, vreg, 0)

    lax.fori_loop(0, CHUNK // WA, win, 0)
    pltpu.sync_copy(hist_v, hist_hbm.at[wid])


@functools.partial(
    pl.kernel,
    mesh=_mesh,
    compiler_params=_CP,
    out_type=[
        jax.ShapeDtypeStruct((ET + BINPAD,), jnp.int32),
        jax.ShapeDtypeStruct((ET + BINPAD,), jnp.float32),
        jax.ShapeDtypeStruct((NBP + 16,), jnp.int32),
    ],
    scratch_types=[
        pltpu.VMEM((NT, NBP), jnp.int32),
        pltpu.VMEM((NBP,), jnp.int32),
        pltpu.VMEM((NBP + 16,), jnp.int32),
        pltpu.VMEM((16,), jnp.float32),
        pltpu.VMEM((WA,), jnp.int32),
        pltpu.VMEM((WA,), jnp.int32),
        pltpu.VMEM((WA,), jnp.float32),
        pltpu.VMEM((WAP,), jnp.int32),
        pltpu.VMEM((WAP,), jnp.float32),
        pltpu.VMEM((WAP,), jnp.int32),
    ],
)
def _binscatter_kernel(esrc_hbm, edst_hbm, alpha_hbm, beta_hbm, hist_hbm,
                       blidx_hbm, bval_hbm, base_hbm,
                       hist_v, ofs_v, base_v, beta_v,
                       src_v, dst_v, a_v, lidx_s, val_s, pos_s):
    wid = lax.axis_index("s") * 2 + lax.axis_index("c")
    m = lax.shift_right_logical(wid, 3)
    off = wid * CHUNK
    pltpu.sync_copy(hist_hbm, hist_v)
    pltpu.sync_copy(beta_hbm, beta_v)

    carry = jnp.int32(0)
    for j in range(NBP // 16):
        tot = jnp.zeros((16,), jnp.int32)
        pre = jnp.zeros((16,), jnp.int32)
        for t in range(NT):
            row = hist_v[t, pl.ds(j * 16, 16)]
            mult = jnp.where(jnp.int32(t) < wid, jnp.int32(1), jnp.int32(0))
            pre = pre + row * mult
            tot = tot + row
        inc = plsc.cumsum(tot)
        excl = inc - tot + carry
        ofs_v[pl.ds(j * 16, 16)] = excl + pre
        base_v[pl.ds(j * 16, 16)] = excl
        carry = carry + jnp.sum(tot)
    base_v[pl.ds(NBP, 16)] = jnp.full((16,), jnp.int32(ET))

    @pl.when(wid == 0)
    def _():
        pltpu.sync_copy(base_v, base_hbm)

    bm = plsc.load_gather(beta_v, [jnp.full((16,), m, jnp.int32)])

    for c in range((WAP - WA) // 16):
        o = WA + c * 16
        padpos = ET + wid * (WAP - WA) + c * 16 + _iota16()
        pos_s[pl.ds(o, 16)] = padpos
        lidx_s[pl.ds(o, 16)] = jnp.zeros((16,), jnp.int32)
        val_s[pl.ds(o, 16)] = jnp.zeros((16,), jnp.float32)

    def win(w, _):
        wo = pl.multiple_of(off + w * WA, 8)
        pltpu.sync_copy(esrc_hbm.at[pl.ds(wo, WA)], src_v)
        pltpu.sync_copy(edst_hbm.at[pl.ds(wo, WA)], dst_v)
        pltpu.sync_copy(alpha_hbm.at[pl.ds(wo, WA)], a_v)

        def vreg(j, _):
            jo = pl.multiple_of(j * 16, 16)
            s = src_v[pl.ds(jo, 16)]
            dv = dst_v[pl.ds(jo, 16)]
            av = a_v[pl.ds(jo, 16)]
            b = lax.shift_right_logical(s, 6)
            cnt, last = plsc.scan_count(b)
            cnt = cnt.astype(jnp.int32)
            ofs = plsc.load_gather(ofs_v, [b])
            pos = ofs + cnt - 1
            plsc.addupdate_scatter(ofs_v, [b], cnt, mask=last)
            lidx = (s & 63) * N + dv
            val = av * bm
            lidx_s[pl.ds(jo, 16)] = lidx
            val_s[pl.ds(jo, 16)] = val
            pos_s[pl.ds(jo, 16)] = pos
            return 0

        lax.fori_loop(0, WA // 16, vreg, 0)
        pltpu.sync_copy(lidx_s, blidx_hbm.at[pos_s])
        pltpu.sync_copy(val_s, bval_hbm.at[pos_s])
        return 0

    lax.fori_loop(0, CHUNK // WA, win, 0)


def _scalar_at(ref, i):
    v = plsc.load_gather(ref, [jnp.full((16,), i, jnp.int32)])
    return jnp.max(v)


@functools.partial(
    pl.kernel,
    mesh=_mesh,
    compiler_params=_CP,
    out_type=jax.ShapeDtypeStruct((N * N,), jnp.float32),
    scratch_types=[
        pltpu.VMEM_SHARED((RWORDS + 16,), jnp.float32),
        pltpu.VMEM((SLICE,), jnp.float32),
        pltpu.VMEM((NBP + 16,), jnp.int32),
        pltpu.VMEM((WBUF,), jnp.int32),
        pltpu.VMEM((WBUF,), jnp.float32),
        pltpu.VMEM((WBUF,), jnp.int32),
        pltpu.VMEM((SLICE,), jnp.float32),
    ],
)
def _accum_kernel(blidx_hbm, bval_hbm, base_hbm, atten_hbm,
                  region, zeros_v, base_v, lbuf, vbuf, idx_s, stage_v):
    cid = lax.axis_index("c")
    sid = lax.axis_index("s")
    pltpu.sync_copy(base_hbm, base_v)

    zero16 = jnp.zeros((16,), jnp.float32)

    def zinit(j, _):
        zeros_v[pl.ds(pl.multiple_of(j * 16, 16), 16)] = zero16
        return 0

    lax.fori_loop(0, SLICE // 16, zinit, 0)
    trash = jnp.int32(RWORDS) + sid
    iota = _iota16()

    def bucket(i, _):
        b = 2 * i + cid
        start = _scalar_at(base_v, b)
        end = _scalar_at(base_v, b + 1)
        myofs = pl.multiple_of(sid * SLICE, 8)
        pltpu.sync_copy(zeros_v, region.at[pl.ds(myofs, SLICE)])
        plsc.subcore_barrier()

        def w_cond(w):
            return start + w * WB < end

        def w_body(w):
            s = start + w * WB
            s0 = pl.multiple_of(s & jnp.int32(-8), 8)
            pltpu.sync_copy(blidx_hbm.at[pl.ds(s0, WBUF)], lbuf)
            pltpu.sync_copy(bval_hbm.at[pl.ds(s0, WBUF)], vbuf)
            hi = jnp.minimum(s + WB, end)
            for c in range(WBUF // 16):
                o = c * 16
                li = lbuf[pl.ds(o, 16)]
                gpos = s0 + o + iota
                valid = (gpos >= s) & (gpos < hi)
                idx_s[pl.ds(o, 16)] = jnp.where(valid, li, trash)
            pltpu.sync_copy(vbuf, region.at[idx_s], add=True)
            return w + 16

        lax.while_loop(w_cond, w_body, sid)
        plsc.subcore_barrier()
        dst = pl.multiple_of(b * RWORDS + sid * SLICE, 8)

        @pl.when(dst + SLICE <= N * N)
        def _():
            pltpu.sync_copy(region.at[pl.ds(myofs, SLICE)], stage_v)
            pltpu.sync_copy(stage_v, atten_hbm.at[pl.ds(dst, SLICE)])

        return 0

    lax.fori_loop(0, (NB + 1) // 2, bucket, 0)


def kernel(z, alpha, edge_index, W1, b1, W2):
    beta = _compute_beta(z, W1, b1, W2)
    z_out = _compute_zout(z, beta)
    beta16 = jnp.pad(beta.reshape(M), (0, 16 - M))
    edge_index = edge_index.astype(jnp.int32)
    esrc = edge_index[:, 0, :].reshape(ET)
    edst = edge_index[:, 1, :].reshape(ET)
    aflat = alpha.reshape(ET)
    hist = _count_kernel(esrc)
    blidx, bval, base = _binscatter_kernel(esrc, edst, aflat, beta16, hist)
    atten = _accum_kernel(blidx, bval, base)
    return (z_out, atten.reshape(N, N))


# speedup vs baseline: 3.1200x; 3.1200x over previous
import functools

import jax
import jax.numpy as jnp
from jax import lax
from jax.experimental import pallas as pl
from jax.experimental.pallas import tpu as pltpu
from jax.experimental.pallas import tpu_sc as plsc

N = 10000
M = 4
D = 128
E = 320000
ET = M * E

NT = 32
CHUNK = ET // NT
BROWS = 64
NB = 157
NBP = 160
RWORDS = BROWS * N
SLICE = RWORDS // 16
BINPAD = 1024
WA = 2000
WB = 496
WBUF = 512

_mesh = plsc.VectorSubcoreMesh(core_axis_name="c", subcore_axis_name="s")
_CP = pltpu.CompilerParams(needs_layout_passes=False)

BNM = 2000
BN2 = 1000


def _iota16():
    return lax.broadcasted_iota(jnp.int32, (16,), 0)


def _proj_kernel(z_ref, w1_ref, b1_ref, w2_ref, wsum_ref, beta_ref, *, ng):
    i = pl.program_id(0)
    h = jnp.dot(z_ref[...], w1_ref[...], preferred_element_type=jnp.float32)
    h = h + b1_ref[...]
    w = jnp.dot(h, w2_ref[...], preferred_element_type=jnp.float32)
    w = jnp.where(w >= 0, w, 0.01 * w)
    rid = lax.broadcasted_iota(jnp.int32, (BNM, M), 0)
    cid = lax.broadcasted_iota(jnp.int32, (BNM, M), 1)
    onehot = (rid % M == cid).astype(jnp.float32)
    part = jnp.sum(w * onehot, axis=0, keepdims=True)

    @pl.when(i == 0)
    def _():
        wsum_ref[...] = part

    @pl.when(i > 0)
    def _():
        wsum_ref[...] = wsum_ref[...] + part

    @pl.when(i == ng - 1)
    def _():
        acc = wsum_ref[...] / float(N)
        mx = jnp.max(acc)
        e = jnp.exp(acc - mx)
        beta_ref[...] = e / jnp.sum(e)


def _compute_beta(z, W1, b1, W2):
    zf = z.reshape(N * M, D)
    ng = (N * M) // BNM
    _, beta = pl.pallas_call(
        functools.partial(_proj_kernel, ng=ng),
        grid=(ng,),
        in_specs=[
            pl.BlockSpec((BNM, D), lambda i: (i, 0)),
            pl.BlockSpec((D, D), lambda i: (0, 0)),
            pl.BlockSpec((1, D), lambda i: (0, 0)),
            pl.BlockSpec((D, 1), lambda i: (0, 0)),
        ],
        out_specs=[
            pl.BlockSpec((1, M), lambda i: (0, 0)),
            pl.BlockSpec((1, M), lambda i: (0, 0)),
        ],
        out_shape=[
            jax.ShapeDtypeStruct((1, M), jnp.float32),
            jax.ShapeDtypeStruct((1, M), jnp.float32),
        ],
    )(zf, W1, b1.reshape(1, D), W2)
    return beta


def _zout_kernel(z_ref, beta_ref, out_ref):
    acc = beta_ref[0, 0] * z_ref[:, 0, :]
    for m in range(1, M):
        acc = acc + beta_ref[0, m] * z_ref[:, m, :]
    out_ref[...] = acc


def _compute_zout(z, beta):
    ng = N // BN2
    return pl.pallas_call(
        _zout_kernel,
        grid=(ng,),
        in_specs=[
            pl.BlockSpec((BN2, M, D), lambda i: (i, 0, 0)),
            pl.BlockSpec((1, M), lambda i: (0, 0)),
        ],
        out_specs=pl.BlockSpec((BN2, D), lambda i: (i, 0)),
        out_shape=jax.ShapeDtypeStruct((N, D), jnp.float32),
    )(z, beta)


@functools.partial(
    pl.kernel,
    mesh=_mesh,
    compiler_params=_CP,
    out_type=[
        jax.ShapeDtypeStruct((ET + BINPAD,), jnp.int32),
        jax.ShapeDtypeStruct((ET + BINPAD,), jnp.float32),
        jax.ShapeDtypeStruct((NT, NBP + 16), jnp.int32),
    ],
    scratch_types=[
        pltpu.VMEM((NBP,), jnp.int32),
        pltpu.VMEM((NBP + 16,), jnp.int32),
        pltpu.VMEM((16,), jnp.float32),
        pltpu.VMEM((WA,), jnp.int32),
        pltpu.VMEM((WA,), jnp.int32),
        pltpu.VMEM((WA,), jnp.float32),
        pltpu.VMEM((CHUNK,), jnp.int32),
        pltpu.VMEM((CHUNK,), jnp.float32),
    ],
)
def _bin_kernel(esrc_hbm, edst_hbm, alpha_hbm, beta_hbm,
                blidx_hbm, bval_hbm, seg_hbm,
                ofs_v, seg_v, beta_v,
                src_v, dst_v, a_v, lidx_bin, val_bin):
    wid = lax.axis_index("s") * 2 + lax.axis_index("c")
    m = lax.shift_right_logical(wid, 3)
    off = wid * CHUNK
    pltpu.sync_copy(beta_hbm, beta_v)
    zero16 = jnp.zeros((16,), jnp.int32)
    for j in range(NBP // 16):
        ofs_v[pl.ds(j * 16, 16)] = zero16

    def cwin(w, _):
        wo = pl.multiple_of(off + w * WA, 8)
        pltpu.sync_copy(esrc_hbm.at[pl.ds(wo, WA)], src_v)

        def vreg(j, _):
            jo = pl.multiple_of(j * 16, 16)
            s = src_v[pl.ds(jo, 16)]
            b = lax.shift_right_logical(s, 6)
            cnt, last = plsc.scan_count(b)
            plsc.addupdate_scatter(ofs_v, [b], cnt.astype(jnp.int32),
                                   mask=last)
            return 0

        return lax.fori_loop(0, WA // 16, vreg, 0)

    lax.fori_loop(0, CHUNK // WA, cwin, 0)

    carry = jnp.int32(0)
    for j in range(NBP // 16):
        tot = ofs_v[pl.ds(j * 16, 16)]
        inc = plsc.cumsum(tot)
        excl = inc - tot + carry
        ofs_v[pl.ds(j * 16, 16)] = excl
        seg_v[pl.ds(j * 16, 16)] = excl + off
        carry = carry + jnp.sum(tot)
    seg_v[pl.ds(NBP, 16)] = jnp.full((16,), jnp.int32(CHUNK)) + off
    pltpu.sync_copy(seg_v, seg_hbm.at[wid])

    bm = plsc.load_gather(beta_v, [jnp.full((16,), m, jnp.int32)])

    def swin(w, _):
        wo = pl.multiple_of(off + w * WA, 8)
        pltpu.sync_copy(esrc_hbm.at[pl.ds(wo, WA)], src_v)
        pltpu.sync_copy(edst_hbm.at[pl.ds(wo, WA)], dst_v)
        pltpu.sync_copy(alpha_hbm.at[pl.ds(wo, WA)], a_v)

        def vreg(j, _):
            jo = pl.multiple_of(j * 16, 16)
            s = src_v[pl.ds(jo, 16)]
            dv = dst_v[pl.ds(jo, 16)]
            av = a_v[pl.ds(jo, 16)]
            b = lax.shift_right_logical(s, 6)
            cnt, last = plsc.scan_count(b)
            cnt = cnt.astype(jnp.int32)
            ofs = plsc.load_gather(ofs_v, [b])
            pos = ofs + cnt - 1
            plsc.addupdate_scatter(ofs_v, [b], cnt, mask=last)
            lidx = (s & 63) * N + dv
            val = av * bm
            plsc.store_scatter(lidx_bin, [pos], lidx)
            plsc.store_scatter(val_bin, [pos], val)
            return 0

        return lax.fori_loop(0, WA // 16, vreg, 0)

    lax.fori_loop(0, CHUNK // WA, swin, 0)
    moff = pl.multiple_of(off, 8)
    pltpu.sync_copy(lidx_bin, blidx_hbm.at[pl.ds(moff, CHUNK)])
    pltpu.sync_copy(val_bin, bval_hbm.at[pl.ds(moff, CHUNK)])


def _scalar_at(ref, i):
    v = plsc.load_gather(ref, [jnp.full((16,), i, jnp.int32)])
    return jnp.max(v)


@functools.partial(
    pl.kernel,
    mesh=_mesh,
    compiler_params=_CP,
    out_type=jax.ShapeDtypeStruct((N * N,), jnp.float32),
    scratch_types=[
        pltpu.VMEM_SHARED((RWORDS + 16,), jnp.float32),
        pltpu.VMEM((SLICE,), jnp.float32),
        pltpu.VMEM((NT, NBP + 16), jnp.int32),
        pltpu.VMEM((WBUF,), jnp.int32),
        pltpu.VMEM((WBUF,), jnp.float32),
        pltpu.VMEM((WBUF,), jnp.int32),
        pltpu.VMEM((SLICE,), jnp.float32),
    ],
)
def _accum_kernel(blidx_hbm, bval_hbm, seg_hbm, atten_hbm,
                  region, zeros_v, seg_v, lbuf, vbuf, idx_s, stage_v):
    cid = lax.axis_index("c")
    sid = lax.axis_index("s")
    pltpu.sync_copy(seg_hbm, seg_v)

    zero16 = jnp.zeros((16,), jnp.float32)

    def zinit(j, _):
        zeros_v[pl.ds(pl.multiple_of(j * 16, 16), 16)] = zero16
        return 0

    lax.fori_loop(0, SLICE // 16, zinit, 0)
    trash = jnp.int32(RWORDS) + sid
    iota = _iota16()

    def bucket(i, _):
        b = 2 * i + cid
        myofs = pl.multiple_of(sid * SLICE, 8)
        pltpu.sync_copy(zeros_v, region.at[pl.ds(myofs, SLICE)])
        plsc.subcore_barrier()

        for q in range(2):
            t = 2 * sid + q
            tv = jnp.full((16,), t, jnp.int32)
            start = jnp.max(plsc.load_gather(seg_v, [tv, jnp.full((16,), b, jnp.int32)]))
            end = jnp.max(plsc.load_gather(seg_v, [tv, jnp.full((16,), b + 1, jnp.int32)]))

            def w_cond(w):
                return start + w * WB < end

            def w_body(w):
                s = start + w * WB
                s0 = pl.multiple_of(s & jnp.int32(-8), 8)
                pltpu.sync_copy(blidx_hbm.at[pl.ds(s0, WBUF)], lbuf)
                pltpu.sync_copy(bval_hbm.at[pl.ds(s0, WBUF)], vbuf)
                hi = jnp.minimum(s + WB, end)
                for c in range(WBUF // 16):
                    o = c * 16
                    li = lbuf[pl.ds(o, 16)]
                    gpos = s0 + o + iota
                    valid = (gpos >= s) & (gpos < hi)
                    idx_s[pl.ds(o, 16)] = jnp.where(valid, li, trash)
                pltpu.sync_copy(vbuf, region.at[idx_s], add=True)
                return w + 1

            lax.while_loop(w_cond, w_body, 0)
        plsc.subcore_barrier()
        dst = pl.multiple_of(b * RWORDS + sid * SLICE, 8)

        @pl.when(dst + SLICE <= N * N)
        def _():
            pltpu.sync_copy(region.at[pl.ds(myofs, SLICE)], stage_v)
            pltpu.sync_copy(stage_v, atten_hbm.at[pl.ds(dst, SLICE)])

        return 0

    lax.fori_loop(0, (NB + 1) // 2, bucket, 0)


def kernel(z, alpha, edge_index, W1, b1, W2):
    beta = _compute_beta(z, W1, b1, W2)
    z_out = _compute_zout(z, beta)
    beta16 = jnp.pad(beta.reshape(M), (0, 16 - M))
    edge_index = edge_index.astype(jnp.int32)
    esrc = edge_index[:, 0, :].reshape(ET)
    edst = edge_index[:, 1, :].reshape(ET)
    aflat = alpha.reshape(ET)
    blidx, bval, seg = _bin_kernel(esrc, edst, aflat, beta16)
    atten = _accum_kernel(blidx, bval, seg)
    return (z_out, atten.reshape(N, N))
